# baseline (speedup 1.0000x reference)
import functools

import jax
import jax.numpy as jnp
from jax import lax
from jax.experimental import pallas as pl
from jax.experimental.pallas import tpu as pltpu
from jax.experimental.pallas import tpu_sc as plsc

_NUM_CORES = 2
_NUM_SUBCORES = 16
_NUM_WORKERS = _NUM_CORES * _NUM_SUBCORES

_C = 16
_LANES = 16


def _emb_kernel(B, S, D,
                ids_hbm, tids_hbm, tok_hbm, pos_hbm, task_hbm, out_hbm,
                idx2d, tidx_v, posbuf, tok_a, tok_b, tok_c, taskbuf,
                sem_tok_a, sem_tok_b, sem_tok_c, sem_pos,
                sem_out_a, sem_out_b, sem_out_c, sem_task):
  srange = S // _NUM_WORKERS
  wid = lax.axis_index("s") * _NUM_CORES + lax.axis_index("c")
  sbase = wid * srange
  chunks_per_b = srange // _C
  nchunks = B * chunks_per_b
  ndg = D // _LANES

  pltpu.sync_copy(ids_hbm.at[pl.ds(sbase, srange)], idx2d.at[0])

  toks = [tok_a, tok_b, tok_c]
  sem_toks = [sem_tok_a, sem_tok_b, sem_tok_c]
  sem_outs = [sem_out_a, sem_out_b, sem_out_c]
  nbuf = len(toks)

  def start_gather(k):
    b, h = k // chunks_per_b, k % chunks_per_b
    return pltpu.async_copy(
        tok_hbm.at[idx2d.at[b, pl.ds(h * _C, _C)]], toks[k % nbuf],
        sem_toks[k % nbuf])

  gcp = [None] * nchunks
  ocp = [None] * nchunks
  gcp[0] = start_gather(0)

  for b in range(1, B):
    pltpu.sync_copy(ids_hbm.at[pl.ds(b * S + sbase, srange)], idx2d.at[b])
  pltpu.sync_copy(tids_hbm, tidx_v)
  taskcp = pltpu.async_copy(task_hbm.at[tidx_v], taskbuf, sem_task)
  poscp = pltpu.async_copy(pos_hbm.at[pl.ds(sbase, srange)], posbuf, sem_pos)

  gcp[1] = start_gather(1)
  taskcp.wait()
  poscp.wait()

  for k in range(nchunks):
    b, h = k // chunks_per_b, k % chunks_per_b
    cur = k % nbuf
    if k + 2 < nchunks:
      if k >= 1:
        ocp[k - 1].wait()
      gcp[k + 2] = start_gather(k + 2)
    gcp[k].wait()
    tok = toks[cur]
    for half in range(2):
      dg0 = half * (ndg // 2)
      tvs = [taskbuf[b, pl.ds((dg0 + dg) * _LANES, _LANES)]
             for dg in range(ndg // 2)]

      def r_body(r, carry, h=h, tok=tok, dg0=dg0, tvs=tvs):
        pr = h * _C + r
        for dg in range(len(tvs)):
          dgs = pl.ds((dg0 + dg) * _LANES, _LANES)
          plsc.addupdate(tok.at[r, dgs], posbuf[pr, dgs] + tvs[dg])
        return carry

      lax.fori_loop(0, _C, r_body, 0)

    ocp[k] = pltpu.async_copy(
        tok, out_hbm.at[pl.ds(b * S + sbase + h * _C, _C)], sem_outs[cur])

  for k in range(max(nchunks - nbuf, 0), nchunks):
    ocp[k].wait()


@jax.jit
def kernel(input_ids, task_ids, token_table, pos_table, task_table):
  B, S = input_ids.shape
  V, D = token_table.shape
  N = B * S
  srange = S // _NUM_WORKERS

  flat_ids = jnp.asarray(input_ids, jnp.int32).reshape(N)
  tids8 = jnp.concatenate([jnp.asarray(task_ids, jnp.int32)] * (8 // B))

  mesh = plsc.VectorSubcoreMesh(core_axis_name="c", subcore_axis_name="s")
  body = functools.partial(_emb_kernel, B, S, D)
  out = pl.kernel(
      body,
      out_type=jax.ShapeDtypeStruct((N, D), jnp.float32),
      mesh=mesh,
      scratch_types=[
          pltpu.VMEM((B, srange), jnp.int32),
          pltpu.VMEM((8,), jnp.int32),
          pltpu.VMEM((srange, D), jnp.float32),
          pltpu.VMEM((_C, D), jnp.float32),
          pltpu.VMEM((_C, D), jnp.float32),
          pltpu.VMEM((_C, D), jnp.float32),
          pltpu.VMEM((8, D), jnp.float32),
          pltpu.SemaphoreType.DMA,
          pltpu.SemaphoreType.DMA,
          pltpu.SemaphoreType.DMA,
          pltpu.SemaphoreType.DMA,
          pltpu.SemaphoreType.DMA,
          pltpu.SemaphoreType.DMA,
          pltpu.SemaphoreType.DMA,
          pltpu.SemaphoreType.DMA,
      ],
  )(flat_ids, tids8, token_table, pos_table, task_table)
  return out.reshape(B, S, D)


# speedup vs baseline: 1.0368x; 1.0368x over previous
import functools

import jax
import jax.numpy as jnp
from jax import lax
from jax.experimental import pallas as pl
from jax.experimental.pallas import tpu as pltpu
from jax.experimental.pallas import tpu_sc as plsc

_NUM_CORES = 2
_NUM_SUBCORES = 16
_NUM_WORKERS = _NUM_CORES * _NUM_SUBCORES

_H = 4
_C = 16
_R = 256


def _gather_body(N, ids_hbm, tok_hbm, out_hbm, idx_all,
                 tok_a, tok_b, tok_c,
                 sem_g_a, sem_g_b, sem_g_c, sem_o_a, sem_o_b, sem_o_c):
  rpw = N // _NUM_WORKERS
  nchunks = rpw // _C
  wid = lax.axis_index("s") * _NUM_CORES + lax.axis_index("c")
  base = wid * rpw
  pltpu.sync_copy(ids_hbm.at[pl.ds(base, rpw)], idx_all)

  toks = [tok_a, tok_b, tok_c]
  sem_gs = [sem_g_a, sem_g_b, sem_g_c]
  sem_os = [sem_o_a, sem_o_b, sem_o_c]
  nbuf = len(toks)

  def start_gather(k):
    return pltpu.async_copy(
        tok_hbm.at[idx_all.at[pl.ds(k * _C, _C)]], toks[k % nbuf],
        sem_gs[k % nbuf])

  gcp = [None] * nchunks
  ocp = [None] * nchunks
  for k in range(min(2, nchunks)):
    gcp[k] = start_gather(k)
  for k in range(nchunks):
    if k + 2 < nchunks:
      if k >= 1:
        ocp[k - 1].wait()
      gcp[k + 2] = start_gather(k + 2)
    gcp[k].wait()
    ocp[k] = pltpu.async_copy(
        toks[k % nbuf], out_hbm.at[pl.ds(base + k * _C, _C)],
        sem_os[k % nbuf])
  for k in range(max(nchunks - nbuf, 0), nchunks):
    ocp[k].wait()


def _sc_gather(ids_flat, token_table):
  N = ids_flat.shape[0]
  D = token_table.shape[1]
  mesh = plsc.VectorSubcoreMesh(core_axis_name="c", subcore_axis_name="s")
  return pl.kernel(
      functools.partial(_gather_body, N),
      out_type=jax.ShapeDtypeStruct((N, D), jnp.float32),
      mesh=mesh,
      scratch_types=[
          pltpu.VMEM((N // _NUM_WORKERS,), jnp.int32),
          pltpu.VMEM((_C, D), jnp.float32),
          pltpu.VMEM((_C, D), jnp.float32),
          pltpu.VMEM((_C, D), jnp.float32),
          pltpu.SemaphoreType.DMA,
          pltpu.SemaphoreType.DMA,
          pltpu.SemaphoreType.DMA,
          pltpu.SemaphoreType.DMA,
          pltpu.SemaphoreType.DMA,
          pltpu.SemaphoreType.DMA,
      ],
  )(ids_flat, token_table)


def _add_body(W, tids_ref, *refs):
  g_ref, pos_ref, tt_ref, out_ref = refs[-4:]
  i = pl.program_id(0)
  b = i // (W // _R)
  tid = tids_ref[b]
  trow = tt_ref[pl.ds(tid, 1), :]
  out_ref[...] = g_ref[...] + pos_ref[...] + trow


def _tc_add(j, acc, g_j, pos_table, task_table, task_ids, B, S, D):
  W = S // _H
  nblk = B * W // _R
  blocks_per_b = W // _R
  body = functools.partial(_add_body, W)
  acc_spec = [] if acc is None else [pl.BlockSpec(memory_space=pl.ANY)]
  acc_arg = () if acc is None else (acc,)
  grid_spec = pltpu.PrefetchScalarGridSpec(
      num_scalar_prefetch=1,
      grid=(nblk,),
      in_specs=acc_spec + [
          pl.BlockSpec((_R, D), lambda i, t: (i, 0)),
          pl.BlockSpec((_R, D),
                       lambda i, t: (j * W // _R + i % blocks_per_b, 0)),
          pl.BlockSpec((task_table.shape[0], D), lambda i, t: (0, 0)),
      ],
      out_specs=pl.BlockSpec(
          (_R, D),
          lambda i, t: ((i // blocks_per_b) * (S // _R) + j * blocks_per_b
                        + i % blocks_per_b, 0)),
  )
  return pl.pallas_call(
      body,
      grid_spec=grid_spec,
      out_shape=jax.ShapeDtypeStruct((B * S, D), jnp.float32),
      input_output_aliases={1: 0} if acc is not None else {},
  )(task_ids, *acc_arg, g_j, pos_table, task_table)


@jax.jit
def kernel(input_ids, task_ids, token_table, pos_table, task_table):
  B, S = input_ids.shape
  V, D = token_table.shape
  W = S // _H

  ids = jnp.asarray(input_ids, jnp.int32)
  tids = jnp.asarray(task_ids, jnp.int32)

  gs = [_sc_gather(ids[:, j * W:(j + 1) * W].reshape(B * W), token_table)
        for j in range(_H)]

  acc = None
  for j in range(_H):
    acc = _tc_add(j, acc, gs[j], pos_table, task_table, tids, B, S, D)
  return acc.reshape(B, S, D)


# speedup vs baseline: 1.1603x; 1.1191x over previous
import functools

import jax
import jax.numpy as jnp
from jax import lax
from jax.experimental import pallas as pl
from jax.experimental.pallas import tpu as pltpu
from jax.experimental.pallas import tpu_sc as plsc

_NUM_CORES = 2
_NUM_SUBCORES = 16
_NUM_WORKERS = _NUM_CORES * _NUM_SUBCORES

_H = 4
_C = 32
_R = 512


def _gather_body(N, ids_hbm, tok_hbm, out_hbm, idx_all,
                 tok_a, tok_b, tok_c,
                 sem_g_a, sem_g_b, sem_g_c, sem_o_a, sem_o_b, sem_o_c):
  rpw = N // _NUM_WORKERS
  nchunks = rpw // _C
  wid = lax.axis_index("s") * _NUM_CORES + lax.axis_index("c")
  base = wid * rpw
  pltpu.sync_copy(ids_hbm.at[pl.ds(base, rpw)], idx_all)

  toks = [tok_a, tok_b, tok_c]
  sem_gs = [sem_g_a, sem_g_b, sem_g_c]
  sem_os = [sem_o_a, sem_o_b, sem_o_c]
  nbuf = len(toks)
  assert nchunks <= nbuf

  gcp = [pltpu.async_copy(
      tok_hbm.at[idx_all.at[pl.ds(k * _C, _C)]], toks[k], sem_gs[k])
      for k in range(nchunks)]
  ocp = []
  for k in range(nchunks):
    gcp[k].wait()
    ocp.append(pltpu.async_copy(
        toks[k], out_hbm.at[pl.ds(base + k * _C, _C)], sem_os[k]))
  for cp in ocp:
    cp.wait()


def _sc_gather(ids_flat, token_table):
  N = ids_flat.shape[0]
  D = token_table.shape[1]
  mesh = plsc.VectorSubcoreMesh(core_axis_name="c", subcore_axis_name="s")
  return pl.kernel(
      functools.partial(_gather_body, N),
      out_type=jax.ShapeDtypeStruct((N, D), jnp.float32),
      mesh=mesh,
      scratch_types=[
          pltpu.VMEM((N // _NUM_WORKERS,), jnp.int32),
          pltpu.VMEM((_C, D), jnp.float32),
          pltpu.VMEM((_C, D), jnp.float32),
          pltpu.VMEM((_C, D), jnp.float32),
          pltpu.SemaphoreType.DMA,
          pltpu.SemaphoreType.DMA,
          pltpu.SemaphoreType.DMA,
          pltpu.SemaphoreType.DMA,
          pltpu.SemaphoreType.DMA,
          pltpu.SemaphoreType.DMA,
      ],
  )(ids_flat, token_table)


def _add_body(W, tids_ref, *refs):
  g_ref, pos_ref, tt_ref, out_ref = refs[-4:]
  i = pl.program_id(0)
  b = i // (W // _R)
  tid = tids_ref[b]
  trow = tt_ref[pl.ds(tid, 1), :]
  out_ref[...] = g_ref[...] + pos_ref[...] + trow


def _tc_add(j, acc, g_j, pos_table, task_table, task_ids, B, S, D):
  W = S // _H
  nblk = B * W // _R
  blocks_per_b = W // _R
  body = functools.partial(_add_body, W)
  acc_spec = [] if acc is None else [pl.BlockSpec(memory_space=pl.ANY)]
  acc_arg = () if acc is None else (acc,)
  grid_spec = pltpu.PrefetchScalarGridSpec(
      num_scalar_prefetch=1,
      grid=(nblk,),
      in_specs=acc_spec + [
          pl.BlockSpec((_R, D), lambda i, t: (i, 0)),
          pl.BlockSpec((_R, D),
                       lambda i, t: (j * W // _R + i % blocks_per_b, 0)),
          pl.BlockSpec((task_table.shape[0], D), lambda i, t: (0, 0)),
      ],
      out_specs=pl.BlockSpec(
          (_R, D),
          lambda i, t: ((i // blocks_per_b) * (S // _R) + j * blocks_per_b
                        + i % blocks_per_b, 0)),
  )
  return pl.pallas_call(
      body,
      grid_spec=grid_spec,
      out_shape=jax.ShapeDtypeStruct((B * S, D), jnp.float32),
      input_output_aliases={1: 0} if acc is not None else {},
  )(task_ids, *acc_arg, g_j, pos_table, task_table)


@jax.jit
def kernel(input_ids, task_ids, token_table, pos_table, task_table):
  B, S = input_ids.shape
  V, D = token_table.shape
  W = S // _H

  ids = jnp.asarray(input_ids, jnp.int32)
  tids = jnp.asarray(task_ids, jnp.int32)

  gs = [_sc_gather(ids[:, j * W:(j + 1) * W].reshape(B * W), token_table)
        for j in range(_H)]

  acc = None
  for j in range(_H):
    acc = _tc_add(j, acc, gs[j], pos_table, task_table, tids, B, S, D)
  return acc.reshape(B, S, D)
